# baseline (speedup 1.0000x reference)
import functools

import jax
import jax.numpy as jnp
from jax import lax
from jax.experimental import pallas as pl
from jax.experimental.pallas import tpu as pltpu
from jax.experimental.pallas import tpu_sc as plsc

_B = 16384
_D = 128
_NW = 32
_EPW = _B // _NW
_N_TAG = 1000448
_L = 16

_mesh = plsc.VectorSubcoreMesh(core_axis_name="c", subcore_axis_name="s")


@functools.partial(
    pl.kernel,
    out_type=(
        jax.ShapeDtypeStruct((_B, _D), jnp.float32),
        jax.ShapeDtypeStruct((_B,), jnp.float32),
    ),
    mesh=_mesh,
    scratch_types=[
        pltpu.VMEM((_B,), jnp.int32),
        pltpu.VMEM((_B,), jnp.int32),
        pltpu.VMEM((_EPW,), jnp.int32),
        pltpu.VMEM((_EPW,), jnp.int32),
        pltpu.VMEM((_L,), jnp.int32),
        pltpu.VMEM((_L, _D), jnp.float32),
        pltpu.VMEM((_EPW,), jnp.float32),
        pltpu.VMEM_SHARED((_N_TAG,), jnp.int32),
    ],
)
def _sc_mem(nids_h, jvals_h, new_mem, new_lu, out_mem, out_lu,
            idx_v, val_v, w_v, exp_v, cnt_v, fix_v, lu_v, tag_sh):
    c = lax.axis_index("c")
    s = lax.axis_index("s")
    wid = c * 16 + s
    base = wid * _EPW

    @pl.when(s == 0)
    def _build_tag():
        pltpu.sync_copy(nids_h, idx_v)
        pltpu.sync_copy(jvals_h, val_v)
        pltpu.sync_copy(val_v, tag_sh.at[idx_v])

    @pl.when(s != 0)
    def _stage_slice():
        pltpu.sync_copy(nids_h.at[pl.ds(base, _EPW)],
                        idx_v.at[pl.ds(base, _EPW)])

    pltpu.sync_copy(new_mem.at[pl.ds(base, _EPW)],
                    out_mem.at[pl.ds(base, _EPW)])

    plsc.subcore_barrier()

    pltpu.sync_copy(tag_sh.at[idx_v.at[pl.ds(base, _EPW)]], w_v)
    pltpu.sync_copy(jvals_h.at[pl.ds(base, _EPW)], exp_v)
    pltpu.sync_copy(new_lu.at[w_v], lu_v)
    pltpu.sync_copy(lu_v, out_lu.at[pl.ds(base, _EPW)])

    for q in range(_EPW // _L):
        w = w_v[pl.ds(q * _L, _L)]
        e = exp_v[pl.ds(q * _L, _L)]
        d = jnp.abs(w - e)
        tot = d[0]
        for l in range(1, _L):
            tot = tot + d[l]
        need_fix = tot != 0

        @pl.when(need_fix)
        def _fix(q=q):
            pltpu.sync_copy(new_mem.at[w_v.at[pl.ds(q * _L, _L)]], fix_v)
            pltpu.sync_copy(fix_v, out_mem.at[pl.ds(base + q * _L, _L)])


def kernel(memory, last_update, nids, new_memory, new_last_update):
    del memory, last_update
    jvals = jnp.arange(_B, dtype=jnp.int32)
    return _sc_mem(nids, jvals, new_memory, new_last_update)


# speedup vs baseline: 5.9431x; 5.9431x over previous
import functools

import jax
import jax.numpy as jnp
from jax import lax
from jax.experimental import pallas as pl
from jax.experimental.pallas import tpu as pltpu
from jax.experimental.pallas import tpu_sc as plsc

_B = 16384
_D = 128
_NW = 32
_EPW = _B // _NW
_N_TAG = 1000448
_L = 16

_mesh = plsc.VectorSubcoreMesh(core_axis_name="c", subcore_axis_name="s")


@functools.partial(
    pl.kernel,
    out_type=(
        jax.ShapeDtypeStruct((_B, _D), jnp.float32),
        jax.ShapeDtypeStruct((_B,), jnp.float32),
    ),
    mesh=_mesh,
    scratch_types=[
        pltpu.VMEM((_B,), jnp.int32),
        pltpu.VMEM((_B,), jnp.int32),
        pltpu.VMEM((_EPW,), jnp.int32),
        pltpu.VMEM((_EPW,), jnp.int32),
        pltpu.VMEM((2, 128, _D), jnp.float32),
        pltpu.VMEM((_EPW,), jnp.float32),
        pltpu.VMEM_SHARED((_N_TAG,), jnp.int32),
        pltpu.SemaphoreType.DMA,
        pltpu.SemaphoreType.DMA,
    ],
)
def _sc_mem(nids_h, jvals_h, new_mem, new_lu, out_mem, out_lu,
            idx_v, val_v, w_v, exp_v, rows_v, lu_v, tag_sh, sem_a, sem_b):
    c = lax.axis_index("c")
    s = lax.axis_index("s")
    wid = c * 16 + s
    base = wid * _EPW

    @pl.when(s == 0)
    def _build_tag():
        pltpu.sync_copy(nids_h, idx_v)
        pltpu.sync_copy(jvals_h, val_v)
        pltpu.sync_copy(val_v, tag_sh.at[idx_v])

    @pl.when(s != 0)
    def _stage_slice():
        pltpu.sync_copy(nids_h.at[pl.ds(base, _EPW)],
                        idx_v.at[pl.ds(base, _EPW)])

    sems = (sem_a, sem_b)
    pending = [None, None]
    for q in range(4):
        buf = q % 2
        pending[buf] = pltpu.async_copy(
            new_mem.at[pl.ds(base + q * 128, 128)], rows_v.at[buf],
            sems[buf])
        if q >= 1:
            prev = (q - 1) % 2
            pending[prev].wait()
            pltpu.sync_copy(rows_v.at[prev],
                            out_mem.at[pl.ds(base + (q - 1) * 128, 128)])
    pending[1].wait()
    pltpu.sync_copy(rows_v.at[1], out_mem.at[pl.ds(base + 3 * 128, 128)])

    plsc.subcore_barrier()

    pltpu.sync_copy(tag_sh.at[idx_v.at[pl.ds(base, _EPW)]], w_v)
    pltpu.sync_copy(jvals_h.at[pl.ds(base, _EPW)], exp_v)
    pltpu.sync_copy(new_lu.at[w_v], lu_v)
    pltpu.sync_copy(lu_v, out_lu.at[pl.ds(base, _EPW)])

    for q in range(_EPW // _L):
        w = w_v[pl.ds(q * _L, _L)]
        e = exp_v[pl.ds(q * _L, _L)]
        d = jnp.abs(w - e)
        tot = d[0]
        for l in range(1, _L):
            tot = tot + d[l]
        need_fix = tot != 0

        @pl.when(need_fix)
        def _fix(q=q):
            pltpu.sync_copy(new_mem.at[w_v.at[pl.ds(q * _L, _L)]],
                            rows_v.at[0, pl.ds(0, _L)])
            pltpu.sync_copy(rows_v.at[0, pl.ds(0, _L)],
                            out_mem.at[pl.ds(base + q * _L, _L)])


def kernel(memory, last_update, nids, new_memory, new_last_update):
    del memory, last_update
    jvals = jnp.arange(_B, dtype=jnp.int32)
    return _sc_mem(nids, jvals, new_memory, new_last_update)


# speedup vs baseline: 7.9724x; 1.3414x over previous
import functools

import jax
import jax.numpy as jnp
from jax import lax
from jax.experimental import pallas as pl
from jax.experimental.pallas import tpu as pltpu
from jax.experimental.pallas import tpu_sc as plsc

_B = 16384
_D = 128
_NW = 32
_EPW = _B // _NW
_RCH = 128
_NQ = _EPW // _RCH
_N_TAG = 1000448

_mesh = plsc.VectorSubcoreMesh(core_axis_name="c", subcore_axis_name="s")


@functools.partial(
    pl.kernel,
    out_type=(
        jax.ShapeDtypeStruct((_B, _D), jnp.float32),
        jax.ShapeDtypeStruct((_B,), jnp.float32),
    ),
    mesh=_mesh,
    scratch_types=[
        pltpu.VMEM((_B,), jnp.int32),
        pltpu.VMEM((_B,), jnp.int32),
        pltpu.VMEM((_EPW,), jnp.int32),
        pltpu.VMEM((2, _RCH, _D), jnp.float32),
        pltpu.VMEM((_EPW,), jnp.float32),
        pltpu.VMEM_SHARED((_N_TAG,), jnp.int32),
        pltpu.SemaphoreType.DMA,
        pltpu.SemaphoreType.DMA,
    ],
)
def _sc_mem(nids_h, jvals_h, new_mem, new_lu, out_mem, out_lu,
            idx_v, val_v, w_v, rows_v, lu_v, tag_sh, sem_a, sem_b):
    c = lax.axis_index("c")
    s = lax.axis_index("s")
    wid = c * 16 + s
    base = wid * _EPW

    @pl.when(s == 0)
    def _build_tag():
        pltpu.sync_copy(nids_h, idx_v)
        pltpu.sync_copy(jvals_h, val_v)
        pltpu.sync_copy(val_v, tag_sh.at[idx_v])

    @pl.when(s != 0)
    def _stage_slice():
        pltpu.sync_copy(nids_h.at[pl.ds(base, _EPW)],
                        idx_v.at[pl.ds(base, _EPW)])

    plsc.subcore_barrier()

    pltpu.sync_copy(tag_sh.at[idx_v.at[pl.ds(base, _EPW)]], w_v)

    sems = (sem_a, sem_b)
    pending = [None, None]
    for q in range(_NQ):
        buf = q % 2
        pending[buf] = pltpu.async_copy(
            new_mem.at[w_v.at[pl.ds(q * _RCH, _RCH)]], rows_v.at[buf],
            sems[buf])
        if q >= 1:
            prev = (q - 1) % 2
            pending[prev].wait()
            pltpu.sync_copy(rows_v.at[prev],
                            out_mem.at[pl.ds(base + (q - 1) * _RCH, _RCH)])
    last = (_NQ - 1) % 2
    pending[last].wait()
    pltpu.sync_copy(rows_v.at[last],
                    out_mem.at[pl.ds(base + (_NQ - 1) * _RCH, _RCH)])

    pltpu.sync_copy(new_lu.at[w_v], lu_v)
    pltpu.sync_copy(lu_v, out_lu.at[pl.ds(base, _EPW)])


def kernel(memory, last_update, nids, new_memory, new_last_update):
    del memory, last_update
    jvals = jnp.arange(_B, dtype=jnp.int32)
    return _sc_mem(nids, jvals, new_memory, new_last_update)


# speedup vs baseline: 8.2660x; 1.0368x over previous
import functools

import jax
import jax.numpy as jnp
from jax import lax
from jax.experimental import pallas as pl
from jax.experimental.pallas import tpu as pltpu
from jax.experimental.pallas import tpu_sc as plsc

_B = 16384
_D = 128
_NW = 32
_EPW = _B // _NW
_RCH = 128
_NQ = _EPW // _RCH
_N_TAG = 1000448

_mesh = plsc.VectorSubcoreMesh(core_axis_name="c", subcore_axis_name="s")


@functools.partial(
    pl.kernel,
    out_type=(
        jax.ShapeDtypeStruct((_B, _D), jnp.float32),
        jax.ShapeDtypeStruct((_B,), jnp.float32),
    ),
    mesh=_mesh,
    scratch_types=[
        pltpu.VMEM((_B,), jnp.int32),
        pltpu.VMEM((_B,), jnp.int32),
        pltpu.VMEM((_EPW,), jnp.int32),
        pltpu.VMEM((2, _RCH, _D), jnp.float32),
        pltpu.VMEM((_EPW,), jnp.float32),
        pltpu.VMEM_SHARED((_N_TAG,), jnp.int32),
        pltpu.SemaphoreType.DMA,
        pltpu.SemaphoreType.DMA,
        pltpu.SemaphoreType.DMA,
    ],
)
def _sc_mem(nids_h, jvals_h, new_mem, new_lu, out_mem, out_lu,
            idx_v, val_v, w_v, rows_v, lu_v, tag_sh, sem_a, sem_b, sem_c):
    c = lax.axis_index("c")
    s = lax.axis_index("s")
    wid = c * 16 + s
    base = wid * _EPW

    @pl.when(s == 0)
    def _build_tag():
        cp_i = pltpu.async_copy(nids_h, idx_v, sem_a)
        cp_v = pltpu.async_copy(jvals_h, val_v, sem_b)
        cp_i.wait()
        cp_v.wait()
        pltpu.sync_copy(val_v, tag_sh.at[idx_v])

    @pl.when(s != 0)
    def _stage_slice():
        pltpu.sync_copy(nids_h.at[pl.ds(base, _EPW)],
                        idx_v.at[pl.ds(base, _EPW)])

    plsc.subcore_barrier()

    pltpu.sync_copy(tag_sh.at[idx_v.at[pl.ds(base, _EPW)]], w_v)
    lu_cp = pltpu.async_copy(new_lu.at[w_v], lu_v, sem_c)

    sems = (sem_a, sem_b)
    pending = [None, None]
    for q in range(_NQ):
        buf = q % 2
        pending[buf] = pltpu.async_copy(
            new_mem.at[w_v.at[pl.ds(q * _RCH, _RCH)]], rows_v.at[buf],
            sems[buf])
        if q >= 1:
            prev = (q - 1) % 2
            pending[prev].wait()
            pltpu.sync_copy(rows_v.at[prev],
                            out_mem.at[pl.ds(base + (q - 1) * _RCH, _RCH)])
    last = (_NQ - 1) % 2
    pending[last].wait()
    pltpu.sync_copy(rows_v.at[last],
                    out_mem.at[pl.ds(base + (_NQ - 1) * _RCH, _RCH)])

    lu_cp.wait()
    pltpu.sync_copy(lu_v, out_lu.at[pl.ds(base, _EPW)])


def kernel(memory, last_update, nids, new_memory, new_last_update):
    del memory, last_update
    jvals = jnp.arange(_B, dtype=jnp.int32)
    return _sc_mem(nids, jvals, new_memory, new_last_update)
